# Initial kernel scaffold; baseline (speedup 1.0000x reference)
import functools

import jax
import jax.numpy as jnp
from jax import lax
from jax.experimental import pallas as pl
from jax.experimental.pallas import tpu as pltpu
from jax.experimental.pallas import tpu_sc as plsc

ROWS = 4096
COLS = 200
EMBED_DIM = 32
B = ROWS * COLS

_NUM_CORES = 2
_NUM_SUBCORES = 16
NW = _NUM_CORES * _NUM_SUBCORES
B_PER_W = B // NW
CHUNK = 1600
NCHUNK = B_PER_W // CHUNK


def _make_gather():
    mesh = plsc.VectorSubcoreMesh(core_axis_name="c", subcore_axis_name="s")

    @functools.partial(
        pl.kernel,
        mesh=mesh,
        out_type=jax.ShapeDtypeStruct((B, EMBED_DIM), jnp.float32),
        scratch_types=[
            pltpu.VMEM((CHUNK,), jnp.int32),
            pltpu.VMEM((CHUNK, EMBED_DIM), jnp.float32),
            pltpu.SemaphoreType.DMA,
        ],
    )
    def gather_kernel(idx_hbm, table_hbm, out_hbm, idx_v, rows_v, sem):
        wid = lax.axis_index("s") * _NUM_CORES + lax.axis_index("c")
        base = wid * B_PER_W

        def body(i, _):
            off = base + i * CHUNK
            pltpu.sync_copy(idx_hbm.at[pl.ds(off, CHUNK)], idx_v)
            pltpu.async_copy(table_hbm.at[idx_v], rows_v, sem).wait()
            pltpu.sync_copy(rows_v, out_hbm.at[pl.ds(off, CHUNK)])
            return 0

        lax.fori_loop(0, NCHUNK, body, 0)

    return gather_kernel


_gather = _make_gather()


def kernel(sentence, table):
    idx = sentence.reshape(B).astype(jnp.int32)
    out = _gather(idx, table)
    return out.reshape(ROWS, COLS, EMBED_DIM)


# speedup vs baseline: 1.4797x; 1.4797x over previous
import functools

import jax
import jax.numpy as jnp
from jax import lax
from jax.experimental import pallas as pl
from jax.experimental.pallas import tpu as pltpu
from jax.experimental.pallas import tpu_sc as plsc

ROWS = 4096
COLS = 200
EMBED_DIM = 32
B = ROWS * COLS

_NUM_CORES = 2
_NUM_SUBCORES = 16
NW = _NUM_CORES * _NUM_SUBCORES
B_PER_W = B // NW
CHUNK = 1600
NCHUNK = B_PER_W // CHUNK


def _make_gather():
    mesh = plsc.VectorSubcoreMesh(core_axis_name="c", subcore_axis_name="s")

    @functools.partial(
        pl.kernel,
        mesh=mesh,
        out_type=jax.ShapeDtypeStruct((B, EMBED_DIM), jnp.float32),
        compiler_params=pltpu.CompilerParams(use_tc_tiling_on_sc=False),
        scratch_types=[
            pltpu.VMEM((CHUNK,), jnp.int32),
            pltpu.VMEM((CHUNK, EMBED_DIM), jnp.float32),
            pltpu.SemaphoreType.DMA,
        ],
    )
    def gather_kernel(idx_hbm, table_hbm, out_hbm, idx_v, rows_v, sem):
        wid = lax.axis_index("s") * _NUM_CORES + lax.axis_index("c")
        base = wid * B_PER_W

        def body(i, _):
            off = base + i * CHUNK
            pltpu.sync_copy(idx_hbm.at[pl.ds(off, CHUNK)], idx_v)
            pltpu.async_copy(table_hbm.at[idx_v], rows_v, sem).wait()
            pltpu.sync_copy(rows_v, out_hbm.at[pl.ds(off, CHUNK)])
            return 0

        lax.fori_loop(0, NCHUNK, body, 0)

    return gather_kernel


_gather = _make_gather()


def kernel(sentence, table):
    idx = sentence.reshape(B).astype(jnp.int32)
    out = _gather(idx, table)
    return out.reshape(ROWS, COLS, EMBED_DIM)


# speedup vs baseline: 1.4923x; 1.0085x over previous
import functools

import jax
import jax.numpy as jnp
from jax import lax
from jax.experimental import pallas as pl
from jax.experimental.pallas import tpu as pltpu
from jax.experimental.pallas import tpu_sc as plsc

ROWS = 4096
COLS = 200
EMBED_DIM = 32
B = ROWS * COLS

_NUM_CORES = 2
_NUM_SUBCORES = 16
NW = _NUM_CORES * _NUM_SUBCORES
B_PER_W = B // NW
CHUNK = 1600
NCHUNK = B_PER_W // CHUNK
NBUF = 2


def _make_gather():
    mesh = plsc.VectorSubcoreMesh(core_axis_name="c", subcore_axis_name="s")

    @functools.partial(
        pl.kernel,
        mesh=mesh,
        out_type=jax.ShapeDtypeStruct((B, EMBED_DIM), jnp.float32),
        compiler_params=pltpu.CompilerParams(use_tc_tiling_on_sc=False),
        scratch_types=[
            pltpu.VMEM((CHUNK,), jnp.int32),
            pltpu.VMEM((CHUNK,), jnp.int32),
            pltpu.VMEM((CHUNK, EMBED_DIM), jnp.float32),
            pltpu.VMEM((CHUNK, EMBED_DIM), jnp.float32),
            pltpu.SemaphoreType.DMA,
            pltpu.SemaphoreType.DMA,
            pltpu.SemaphoreType.DMA,
        ],
    )
    def gather_kernel(idx_hbm, table_hbm, out_hbm, idx0, idx1, rows0, rows1,
                      sem_g, sem_w0, sem_w1):
        wid = lax.axis_index("s") * _NUM_CORES + lax.axis_index("c")
        base = wid * B_PER_W
        idx_bufs = (idx0, idx1)
        rows_bufs = (rows0, rows1)
        w_sems = (sem_w0, sem_w1)

        def outer(g, _):
            for b in range(NBUF):
                i = g * NBUF + b
                off = base + i * CHUNK
                pltpu.sync_copy(idx_hbm.at[pl.ds(off, CHUNK)], idx_bufs[b])

                @pl.when(g > 0)
                def _wait_prev_writeback():
                    pltpu.make_async_copy(
                        rows_bufs[b], out_hbm.at[pl.ds(base, CHUNK)], w_sems[b]
                    ).wait()

                pltpu.async_copy(
                    table_hbm.at[idx_bufs[b]], rows_bufs[b], sem_g
                ).wait()
                pltpu.async_copy(
                    rows_bufs[b], out_hbm.at[pl.ds(off, CHUNK)], w_sems[b]
                )
            return 0

        lax.fori_loop(0, NCHUNK // NBUF, outer, 0)
        for b in range(NBUF):
            pltpu.make_async_copy(
                rows_bufs[b], out_hbm.at[pl.ds(base, CHUNK)], w_sems[b]
            ).wait()

    return gather_kernel


_gather = _make_gather()


def kernel(sentence, table):
    idx = sentence.reshape(B).astype(jnp.int32)
    out = _gather(idx, table)
    return out.reshape(ROWS, COLS, EMBED_DIM)
